# baseline (speedup 1.0000x reference)
import functools

import jax
import jax.numpy as jnp
from jax import lax
from jax.experimental import pallas as pl
from jax.experimental.pallas import tpu as pltpu
from jax.experimental.pallas import tpu_sc as plsc

LATENT = 64
BATCH = 16384
IDX_CHUNK = 128


def _gmf_sc(nc, ns):
    nw = nc * ns
    bpw = BATCH // nw
    nchunk = bpw // IDX_CHUNK
    ngroup = bpw // 16

    mesh = plsc.VectorSubcoreMesh(core_axis_name="c", subcore_axis_name="s")

    @functools.partial(
        pl.kernel,
        mesh=mesh,
        out_type=jax.ShapeDtypeStruct((BATCH,), jnp.float32),
        compiler_params=pltpu.CompilerParams(use_tc_tiling_on_sc=False),
        scratch_types=[
            pltpu.VMEM((nchunk, IDX_CHUNK), jnp.int32),
            pltpu.VMEM((nchunk, IDX_CHUNK), jnp.int32),
            pltpu.VMEM((bpw, LATENT), jnp.float32),
            pltpu.VMEM((bpw, LATENT), jnp.float32),
            pltpu.VMEM((LATENT,), jnp.float32),
            pltpu.VMEM((16,), jnp.float32),
            pltpu.VMEM((bpw,), jnp.float32),
            pltpu.SemaphoreType.DMA,
        ],
    )
    def gmf(users_hbm, items_hbm, ut_hbm, it_hbm, w_hbm, b_hbm, out_hbm,
            idx_u, idx_i, rows_u, rows_i, w_v, b_v, out_v, sem):
        wid = lax.axis_index("s") * nc + lax.axis_index("c")
        base = wid * bpw

        pltpu.sync_copy(users_hbm.at[wid], idx_u)
        pltpu.sync_copy(items_hbm.at[wid], idx_i)
        pltpu.sync_copy(w_hbm, w_v)
        pltpu.sync_copy(b_hbm, b_v)

        copies = []
        for c in range(nchunk):
            copies.append(pltpu.async_copy(
                ut_hbm.at[idx_u.at[c]],
                rows_u.at[pl.ds(c * IDX_CHUNK, IDX_CHUNK)], sem))
            copies.append(pltpu.async_copy(
                it_hbm.at[idx_i.at[c]],
                rows_i.at[pl.ds(c * IDX_CHUNK, IDX_CHUNK)], sem))
        for cp in copies:
            cp.wait()

        w0 = w_v[pl.ds(0, 16)]
        w1 = w_v[pl.ds(16, 16)]
        w2 = w_v[pl.ds(32, 16)]
        w3 = w_v[pl.ds(48, 16)]
        bias = b_v[...]
        lane = lax.iota(jnp.int32, 16)
        gd = lax.GatherDimensionNumbers(
            offset_dims=(), collapsed_slice_dims=(0,), start_index_map=(0,))

        def vperm(x, idx):
            return lax.gather(x, idx[:, None], gd, slice_sizes=(1,),
                              mode=lax.GatherScatterMode.PROMISE_IN_BOUNDS)

        def hsum_all(p):
            for bit in (8, 4, 2, 1):
                p = p + vperm(p, lane ^ bit)
            return p

        def group(g, carry):
            gbase = g * 16
            acc = jnp.zeros((16,), jnp.float32)
            for jj in range(16):
                j = gbase + jj
                p = (rows_u[j, pl.ds(0, 16)] * rows_i[j, pl.ds(0, 16)] * w0
                     + rows_u[j, pl.ds(16, 16)] * rows_i[j, pl.ds(16, 16)] * w1
                     + rows_u[j, pl.ds(32, 16)] * rows_i[j, pl.ds(32, 16)] * w2
                     + rows_u[j, pl.ds(48, 16)] * rows_i[j, pl.ds(48, 16)] * w3)
                s = hsum_all(p)
                acc = jnp.where(lane == jj, s, acc)
            r = acc + bias
            r = 1.0 / (1.0 + jnp.exp(-r))
            out_v[pl.ds(gbase, 16)] = r
            return carry

        lax.fori_loop(0, ngroup, group, 0)
        pltpu.sync_copy(out_v, out_hbm.at[pl.ds(base, bpw)])

    return gmf


def kernel(users, items, user_table, item_table, W, b):
    info = plsc.get_sparse_core_info()
    nc, ns = info.num_cores, info.num_subcores
    nw = nc * ns
    users_r = users.astype(jnp.int32).reshape(nw, BATCH // nw // IDX_CHUNK,
                                              IDX_CHUNK)
    items_r = items.astype(jnp.int32).reshape(nw, BATCH // nw // IDX_CHUNK,
                                              IDX_CHUNK)
    w_flat = W.reshape(LATENT)
    b16 = jnp.broadcast_to(b, (16,))
    out = _gmf_sc(nc, ns)(users_r, items_r, user_table, item_table,
                          w_flat, b16)
    return out.reshape(BATCH, 1)
